# baseline (speedup 1.0000x reference)
import jax
import jax.numpy as jnp
from jax.experimental import pallas as pl
from jax.experimental.pallas import tpu as pltpu

EPS = 1e-5
LT = 256
HC = 1024


def _make_body(NT, D, H):
    def _body(x_ref, a_ref, b_ref, owt_ref, ob_ref, n1_ref, n2_ref,
              wug_ref, bug_ref, w3t_ref, w3b_ref,
              o_ref, yprev_ref, xprev_ref, carry_ref):
        j = pl.program_id(1)
        slot = jax.lax.rem(j, 2)
        pslot = jax.lax.rem(j + 1, 2)

        yp = yprev_ref[pslot]
        xp = xprev_ref[pslot]
        ssm = jnp.dot(yp, owt_ref[...], preferred_element_type=jnp.float32)
        x1 = xp + ssm + ob_ref[...]
        rms2 = jax.lax.rsqrt(jnp.mean(x1 * x1, axis=-1, keepdims=True) + EPS)
        x1n = ((x1 * rms2) * n2_ref[...]).astype(jnp.bfloat16)
        o = x1
        for hc in range(H // HC):
            sl = slice(hc * 2 * HC, (hc + 1) * 2 * HC)
            z = jnp.dot(x1n, wug_ref[:, sl], preferred_element_type=jnp.float32) \
                + bug_ref[:, sl]
            u = z[:, :HC]
            g = z[:, HC:]
            h = (jax.nn.silu(g) * u).astype(jnp.bfloat16)
            o = o + jnp.dot(h, w3t_ref[pl.ds(hc * HC, HC), :],
                            preferred_element_type=jnp.float32)
        o_ref[0] = o + w3b_ref[...]

        xb = x_ref[0]
        rms1 = jax.lax.rsqrt(jnp.mean(xb * xb, axis=-1, keepdims=True) + EPS)
        y = (xb * rms1) * (n1_ref[...] * b_ref[...])
        at = jnp.tanh(a_ref[...])
        row = jax.lax.broadcasted_iota(jnp.int32, (LT, 1), 0)
        carry = jnp.where(j == 0, jnp.zeros((1, D), jnp.float32),
                          carry_ref[...])
        y = y + jnp.where(row == 0, jnp.float32(1.0), jnp.float32(0.0)) \
            * (at * carry)
        p = at
        k = 1
        while k < LT:
            shifted = jnp.concatenate(
                [jnp.zeros((k, D), jnp.float32), y[: LT - k]], axis=0)
            y = y + p * shifted
            p = p * p
            k *= 2
        carry_ref[...] = y[LT - 1: LT]
        yprev_ref[slot] = y.astype(jnp.bfloat16)
        xprev_ref[slot] = xb

    return _body


def kernel(x, a, b, out_w, out_b, n1_w, n2_w,
           w1_w, w1_b, w2_w, w2_b, w3_w, w3_b):
    B, T, D = x.shape
    H = w1_w.shape[0]
    NT = T // LT
    NHC = H // HC

    owt = out_w.T.astype(jnp.bfloat16)
    w1t = w1_w.T.astype(jnp.bfloat16)
    w2t = w2_w.T.astype(jnp.bfloat16)
    w3t = w3_w.T.astype(jnp.bfloat16)
    wug = jnp.concatenate(
        [w1t.reshape(D, NHC, 1, HC), w2t.reshape(D, NHC, 1, HC)], axis=2
    ).reshape(D, 2 * H)
    bug = jnp.concatenate(
        [w1_b.reshape(NHC, 1, HC), w2_b.reshape(NHC, 1, HC)], axis=1
    ).reshape(1, 2 * H)

    a2 = a.reshape(1, D)
    b2 = b.reshape(1, D)
    ob2 = out_b.reshape(1, D)
    n12 = n1_w.reshape(1, D)
    n22 = n2_w.reshape(1, D)
    w3b2 = w3_b.reshape(1, D)

    vec_d = pl.BlockSpec((1, D), lambda i, j: (0, 0))
    full = lambda shape: pl.BlockSpec(shape, lambda i, j: (0, 0))

    grid = (B, NT + 1)
    out = pl.pallas_call(
        _make_body(NT, D, H),
        grid=grid,
        in_specs=[
            pl.BlockSpec((1, LT, D),
                         lambda i, j: (i, jnp.minimum(j, NT - 1), 0)),
            vec_d,
            vec_d,
            full((D, D)),
            vec_d,
            vec_d,
            vec_d,
            full((D, 2 * H)),
            pl.BlockSpec((1, 2 * H), lambda i, j: (0, 0)),
            full((H, D)),
            vec_d,
        ],
        out_specs=pl.BlockSpec((1, LT, D),
                               lambda i, j: (i, jnp.maximum(j - 1, 0), 0)),
        out_shape=jax.ShapeDtypeStruct((B, T, D), jnp.float32),
        scratch_shapes=[
            pltpu.VMEM((2, LT, D), jnp.bfloat16),
            pltpu.VMEM((2, LT, D), jnp.float32),
            pltpu.VMEM((1, D), jnp.float32),
        ],
        compiler_params=pltpu.CompilerParams(
            dimension_semantics=("parallel", "arbitrary"),
            vmem_limit_bytes=60000 * 1024,
        ),
        name="ssm_block_fused",
    )(x, a2, b2, owt, ob2, n12, n22, wug, bug, w3t, w3b2)
    return out


# speedup vs baseline: 1.1014x; 1.1014x over previous
import jax
import jax.numpy as jnp
from jax.experimental import pallas as pl
from jax.experimental.pallas import tpu as pltpu

EPS = 1e-5
LT = 512
HC = 1024


def _make_body(D, H):
    def _body(x_ref, a_ref, b_ref, owt_ref, ob_ref, n1_ref, n2_ref,
              wug_ref, bug_ref, w3t_ref, w3b_ref, o_ref, carry_ref):
        t = pl.program_id(1)

        @pl.when(t == 0)
        def _():
            carry_ref[...] = jnp.zeros_like(carry_ref)

        xb = x_ref[0]

        rms1 = jax.lax.rsqrt(jnp.mean(xb * xb, axis=-1, keepdims=True) + EPS)
        y = (xb * rms1) * (n1_ref[...] * b_ref[...])

        at = jnp.tanh(a_ref[...])
        row8 = jax.lax.broadcasted_iota(jnp.int32, (8, 1), 0)
        head = y[0:8] + jnp.where(row8 == 0, jnp.float32(1.0),
                                  jnp.float32(0.0)) * (at * carry_ref[...])
        y = jnp.concatenate([head, y[8:]], axis=0)
        p = at
        k = 1
        while k < LT:
            shifted = jnp.concatenate(
                [jnp.zeros((k, D), jnp.float32), y[: LT - k]], axis=0)
            y = y + p * shifted
            p = p * p
            k *= 2
        carry_ref[...] = y[LT - 1: LT]

        ssm = jnp.dot(y.astype(jnp.bfloat16), owt_ref[...],
                      preferred_element_type=jnp.float32)
        x1 = xb + ssm + ob_ref[...]

        rms2 = jax.lax.rsqrt(jnp.mean(x1 * x1, axis=-1, keepdims=True) + EPS)
        x1n = ((x1 * rms2) * n2_ref[...]).astype(jnp.bfloat16)

        o = x1
        for hc in range(H // HC):
            sl = slice(hc * 2 * HC, (hc + 1) * 2 * HC)
            z = jnp.dot(x1n, wug_ref[:, sl], preferred_element_type=jnp.float32) \
                + bug_ref[:, sl]
            u = z[:, :HC]
            g = z[:, HC:]
            h = (jax.nn.silu(g) * u).astype(jnp.bfloat16)
            o = o + jnp.dot(h, w3t_ref[pl.ds(hc * HC, HC), :],
                            preferred_element_type=jnp.float32)
        o_ref[0] = o + w3b_ref[...]

    return _body


def kernel(x, a, b, out_w, out_b, n1_w, n2_w,
           w1_w, w1_b, w2_w, w2_b, w3_w, w3_b):
    B, T, D = x.shape
    H = w1_w.shape[0]
    NT = T // LT
    NHC = H // HC

    owt = out_w.T.astype(jnp.bfloat16)
    w1t = w1_w.T.astype(jnp.bfloat16)
    w2t = w2_w.T.astype(jnp.bfloat16)
    w3t = w3_w.T.astype(jnp.bfloat16)
    wug = jnp.concatenate(
        [w1t.reshape(D, NHC, 1, HC), w2t.reshape(D, NHC, 1, HC)], axis=2
    ).reshape(D, 2 * H)
    bug = jnp.concatenate(
        [w1_b.reshape(NHC, 1, HC), w2_b.reshape(NHC, 1, HC)], axis=1
    ).reshape(1, 2 * H)

    a2 = a.reshape(1, D)
    b2 = b.reshape(1, D)
    ob2 = out_b.reshape(1, D)
    n12 = n1_w.reshape(1, D)
    n22 = n2_w.reshape(1, D)
    w3b2 = w3_b.reshape(1, D)

    vec_d = pl.BlockSpec((1, D), lambda i, j: (0, 0))
    full = lambda shape: pl.BlockSpec(shape, lambda i, j: (0, 0))

    grid = (B, NT)
    out = pl.pallas_call(
        _make_body(D, H),
        grid=grid,
        in_specs=[
            pl.BlockSpec((1, LT, D), lambda i, j: (i, j, 0)),
            vec_d,
            vec_d,
            full((D, D)),
            vec_d,
            vec_d,
            vec_d,
            full((D, 2 * H)),
            pl.BlockSpec((1, 2 * H), lambda i, j: (0, 0)),
            full((H, D)),
            vec_d,
        ],
        out_specs=pl.BlockSpec((1, LT, D), lambda i, j: (i, j, 0)),
        out_shape=jax.ShapeDtypeStruct((B, T, D), jnp.float32),
        scratch_shapes=[pltpu.VMEM((1, D), jnp.float32)],
        compiler_params=pltpu.CompilerParams(
            dimension_semantics=("parallel", "arbitrary"),
            vmem_limit_bytes=60000 * 1024,
        ),
        name="ssm_block_fused",
    )(x, a2, b2, owt, ob2, n12, n22, wug, bug, w3t, w3b2)
    return out


# speedup vs baseline: 1.1037x; 1.0021x over previous
import jax
import jax.numpy as jnp
from jax.experimental import pallas as pl
from jax.experimental.pallas import tpu as pltpu

EPS = 1e-5
LT = 256
HC = 1024


def _make_body(D, H):
    def _body(x_ref, a_ref, b_ref, owt_ref, ob_ref, n1_ref, n2_ref,
              wug_ref, bug_ref, w3t_ref, w3b_ref, o_ref, carry_ref):
        t = pl.program_id(1)

        @pl.when(t == 0)
        def _():
            carry_ref[...] = jnp.zeros_like(carry_ref)

        xb = x_ref[0]

        rms1 = jax.lax.rsqrt(jnp.mean(xb * xb, axis=-1, keepdims=True) + EPS)
        y = (xb * rms1) * (n1_ref[...] * b_ref[...])

        at = jnp.tanh(a_ref[...])
        row8 = jax.lax.broadcasted_iota(jnp.int32, (8, 1), 0)
        head = y[0:8] + jnp.where(row8 == 0, jnp.float32(1.0),
                                  jnp.float32(0.0)) * (at * carry_ref[...])
        y = jnp.concatenate([head, y[8:]], axis=0)
        p = at
        k = 1
        while k < LT:
            shifted = jnp.concatenate(
                [jnp.zeros((k, D), jnp.float32), y[: LT - k]], axis=0)
            y = y + p * shifted
            p = p * p
            k *= 2
        carry_ref[...] = y[LT - 1: LT]

        ssm = jnp.dot(y.astype(jnp.bfloat16), owt_ref[...],
                      preferred_element_type=jnp.float32)
        x1 = xb + ssm + ob_ref[...]

        rms2 = jax.lax.rsqrt(jnp.mean(x1 * x1, axis=-1, keepdims=True) + EPS)
        x1n = ((x1 * rms2) * n2_ref[...]).astype(jnp.bfloat16)

        o = x1
        for hc in range(H // HC):
            sl = slice(hc * 2 * HC, (hc + 1) * 2 * HC)
            z = jnp.dot(x1n, wug_ref[:, sl], preferred_element_type=jnp.float32) \
                + bug_ref[:, sl]
            u = z[:, :HC]
            g = z[:, HC:]
            h = (jax.nn.silu(g) * u).astype(jnp.bfloat16)
            o = o + jnp.dot(h, w3t_ref[pl.ds(hc * HC, HC), :],
                            preferred_element_type=jnp.float32)
        o_ref[0] = o + w3b_ref[...]

    return _body


def kernel(x, a, b, out_w, out_b, n1_w, n2_w,
           w1_w, w1_b, w2_w, w2_b, w3_w, w3_b):
    B, T, D = x.shape
    H = w1_w.shape[0]
    NT = T // LT
    NHC = H // HC

    owt = out_w.T.astype(jnp.bfloat16)
    w1t = w1_w.T.astype(jnp.bfloat16)
    w2t = w2_w.T.astype(jnp.bfloat16)
    w3t = w3_w.T.astype(jnp.bfloat16)
    wug = jnp.concatenate(
        [w1t.reshape(D, NHC, 1, HC), w2t.reshape(D, NHC, 1, HC)], axis=2
    ).reshape(D, 2 * H)
    bug = jnp.concatenate(
        [w1_b.reshape(NHC, 1, HC), w2_b.reshape(NHC, 1, HC)], axis=1
    ).reshape(1, 2 * H)

    a2 = a.reshape(1, D)
    b2 = b.reshape(1, D)
    ob2 = out_b.reshape(1, D)
    n12 = n1_w.reshape(1, D)
    n22 = n2_w.reshape(1, D)
    w3b2 = w3_b.reshape(1, D)

    vec_d = pl.BlockSpec((1, D), lambda i, j: (0, 0))
    full = lambda shape: pl.BlockSpec(shape, lambda i, j: (0, 0))

    grid = (B, NT)
    out = pl.pallas_call(
        _make_body(D, H),
        grid=grid,
        in_specs=[
            pl.BlockSpec((1, LT, D), lambda i, j: (i, j, 0)),
            vec_d,
            vec_d,
            full((D, D)),
            vec_d,
            vec_d,
            vec_d,
            full((D, 2 * H)),
            pl.BlockSpec((1, 2 * H), lambda i, j: (0, 0)),
            full((H, D)),
            vec_d,
        ],
        out_specs=pl.BlockSpec((1, LT, D), lambda i, j: (i, j, 0)),
        out_shape=jax.ShapeDtypeStruct((B, T, D), jnp.float32),
        scratch_shapes=[pltpu.VMEM((1, D), jnp.float32)],
        compiler_params=pltpu.CompilerParams(
            dimension_semantics=("parallel", "arbitrary"),
            vmem_limit_bytes=60000 * 1024,
        ),
        name="ssm_block_fused",
    )(x, a2, b2, owt, ob2, n12, n22, wug, bug, w3t, w3b2)
    return out


# speedup vs baseline: 1.3950x; 1.2639x over previous
import jax
import jax.numpy as jnp
from jax.experimental import pallas as pl
from jax.experimental.pallas import tpu as pltpu

EPS = 1e-5
LT = 256
HC = 1024

_TRANS_B = (((1,), (1,)), ((), ()))


def _dot_t(lhs, rhs):
    return jax.lax.dot_general(lhs, rhs, _TRANS_B,
                               preferred_element_type=jnp.float32)


def _make_body(D, H):
    def _body(x_ref, a_ref, b_ref, ow_ref, ob_ref, n1_ref, n2_ref,
              w1_ref, w2_ref, w3_ref, w1b_ref, w2b_ref, w3b_ref,
              o_ref, carry_ref):
        t = pl.program_id(1)

        @pl.when(t == 0)
        def _():
            carry_ref[...] = jnp.zeros_like(carry_ref)

        xb = x_ref[0]

        rms1 = jax.lax.rsqrt(jnp.mean(xb * xb, axis=-1, keepdims=True) + EPS)
        y = (xb * rms1) * (n1_ref[...] * b_ref[...])

        at = jnp.tanh(a_ref[...])
        row8 = jax.lax.broadcasted_iota(jnp.int32, (8, 1), 0)
        head = y[0:8] + jnp.where(row8 == 0, jnp.float32(1.0),
                                  jnp.float32(0.0)) * (at * carry_ref[...])
        y = jnp.concatenate([head, y[8:]], axis=0)
        p = at
        k = 1
        while k < LT:
            shifted = jnp.concatenate(
                [jnp.zeros((k, D), jnp.float32), y[: LT - k]], axis=0)
            y = y + p * shifted
            p = p * p
            k *= 2
        carry_ref[...] = y[LT - 1: LT]

        ssm = _dot_t(y.astype(jnp.bfloat16), ow_ref[...])
        x1 = xb + ssm + ob_ref[...]

        rms2 = jax.lax.rsqrt(jnp.mean(x1 * x1, axis=-1, keepdims=True) + EPS)
        x1n = ((x1 * rms2) * n2_ref[...]).astype(jnp.bfloat16)

        o = x1
        for hc in range(H // HC):
            rs = pl.ds(hc * HC, HC)
            cs = pl.ds(hc * HC, HC)
            u = _dot_t(x1n, w1_ref[rs, :]) + w1b_ref[:, cs]
            g = _dot_t(x1n, w2_ref[rs, :]) + w2b_ref[:, cs]
            h = (jax.nn.silu(g) * u).astype(jnp.bfloat16)
            o = o + _dot_t(h, w3_ref[:, cs])
        o_ref[0] = o + w3b_ref[...]

    return _body


def kernel(x, a, b, out_w, out_b, n1_w, n2_w,
           w1_w, w1_b, w2_w, w2_b, w3_w, w3_b):
    B, T, D = x.shape
    H = w1_w.shape[0]
    NT = T // LT

    ow16 = out_w.astype(jnp.bfloat16)
    w116 = w1_w.astype(jnp.bfloat16)
    w216 = w2_w.astype(jnp.bfloat16)
    w316 = w3_w.astype(jnp.bfloat16)

    a2 = a.reshape(1, D)
    b2 = b.reshape(1, D)
    ob2 = out_b.reshape(1, D)
    n12 = n1_w.reshape(1, D)
    n22 = n2_w.reshape(1, D)
    w1b2 = w1_b.reshape(1, H)
    w2b2 = w2_b.reshape(1, H)
    w3b2 = w3_b.reshape(1, D)

    vec_d = pl.BlockSpec((1, D), lambda i, j: (0, 0))
    vec_h = pl.BlockSpec((1, H), lambda i, j: (0, 0))
    full = lambda shape: pl.BlockSpec(shape, lambda i, j: (0, 0))

    grid = (B, NT)
    out = pl.pallas_call(
        _make_body(D, H),
        grid=grid,
        in_specs=[
            pl.BlockSpec((1, LT, D), lambda i, j: (i, j, 0)),
            vec_d,
            vec_d,
            full((D, D)),
            vec_d,
            vec_d,
            vec_d,
            full((H, D)),
            full((H, D)),
            full((D, H)),
            vec_h,
            vec_h,
            vec_d,
        ],
        out_specs=pl.BlockSpec((1, LT, D), lambda i, j: (i, j, 0)),
        out_shape=jax.ShapeDtypeStruct((B, T, D), jnp.float32),
        scratch_shapes=[pltpu.VMEM((1, D), jnp.float32)],
        compiler_params=pltpu.CompilerParams(
            dimension_semantics=("parallel", "arbitrary"),
            vmem_limit_bytes=60000 * 1024,
        ),
        name="ssm_block_fused",
    )(x, a2, b2, ow16, ob2, n12, n22, w116, w216, w316, w1b2, w2b2, w3b2)
    return out


# speedup vs baseline: 1.3990x; 1.0029x over previous
import jax
import jax.numpy as jnp
from jax.experimental import pallas as pl
from jax.experimental.pallas import tpu as pltpu

EPS = 1e-5
LT = 256
HC = 1024
DC = 128

_TRANS_B = (((1,), (1,)), ((), ()))


def _dot_t(lhs, rhs):
    return jax.lax.dot_general(lhs, rhs, _TRANS_B,
                               preferred_element_type=jnp.float32)


def _make_body(D, H):
    def _body(x_ref, a_ref, b_ref, ow_ref, ob_ref, n1_ref, n2_ref,
              w1_ref, w2_ref, w3_ref, w1b_ref, w2b_ref, w3b_ref,
              o_ref, carry_ref):
        t = pl.program_id(1)

        @pl.when(t == 0)
        def _():
            carry_ref[...] = jnp.zeros_like(carry_ref)

        xb = x_ref[0]

        rms1 = jax.lax.rsqrt(jnp.mean(xb * xb, axis=-1, keepdims=True) + EPS)
        y = (xb * rms1) * (n1_ref[...] * b_ref[...])

        at = jnp.tanh(a_ref[...])
        row8 = jax.lax.broadcasted_iota(jnp.int32, (8, 1), 0)
        rmask = jnp.where(row8 == 0, jnp.float32(1.0), jnp.float32(0.0))
        carry = carry_ref[...]
        chain = jnp.zeros((8, DC), jnp.float32)
        parts = []
        for c in range(D // DC):
            cs = slice(c * DC, (c + 1) * DC)
            yc = y[:, cs]
            p = at[:, cs]
            head = yc[0:8] + rmask * (p * carry[:, cs]) + chain
            yc = jnp.concatenate([head, yc[8:]], axis=0)
            k = 1
            while k < LT:
                shifted = jnp.concatenate(
                    [jnp.zeros((k, DC), jnp.float32), yc[: LT - k]], axis=0)
                yc = yc + p * shifted
                p = p * p
                k *= 2
            parts.append(yc)
            chain = yc[LT - 8:] * jnp.float32(0.0)
        y = jnp.concatenate(parts, axis=1)
        carry_ref[...] = y[LT - 1: LT]

        ssm = _dot_t(y.astype(jnp.bfloat16), ow_ref[...])
        x1 = xb + ssm + ob_ref[...]

        rms2 = jax.lax.rsqrt(jnp.mean(x1 * x1, axis=-1, keepdims=True) + EPS)
        x1n = ((x1 * rms2) * n2_ref[...]).astype(jnp.bfloat16)

        o = x1
        for hc in range(H // HC):
            rs = pl.ds(hc * HC, HC)
            cs = pl.ds(hc * HC, HC)
            u = _dot_t(x1n, w1_ref[rs, :]) + w1b_ref[:, cs]
            g = _dot_t(x1n, w2_ref[rs, :]) + w2b_ref[:, cs]
            h = (jax.nn.silu(g) * u).astype(jnp.bfloat16)
            o = o + _dot_t(h, w3_ref[:, cs])
        o_ref[0] = o + w3b_ref[...]

    return _body


def kernel(x, a, b, out_w, out_b, n1_w, n2_w,
           w1_w, w1_b, w2_w, w2_b, w3_w, w3_b):
    B, T, D = x.shape
    H = w1_w.shape[0]
    NT = T // LT

    ow16 = out_w.astype(jnp.bfloat16)
    w116 = w1_w.astype(jnp.bfloat16)
    w216 = w2_w.astype(jnp.bfloat16)
    w316 = w3_w.astype(jnp.bfloat16)

    a2 = a.reshape(1, D)
    b2 = b.reshape(1, D)
    ob2 = out_b.reshape(1, D)
    n12 = n1_w.reshape(1, D)
    n22 = n2_w.reshape(1, D)
    w1b2 = w1_b.reshape(1, H)
    w2b2 = w2_b.reshape(1, H)
    w3b2 = w3_b.reshape(1, D)

    vec_d = pl.BlockSpec((1, D), lambda i, j: (0, 0))
    vec_h = pl.BlockSpec((1, H), lambda i, j: (0, 0))
    full = lambda shape: pl.BlockSpec(shape, lambda i, j: (0, 0))

    grid = (B, NT)
    out = pl.pallas_call(
        _make_body(D, H),
        grid=grid,
        in_specs=[
            pl.BlockSpec((1, LT, D), lambda i, j: (i, j, 0)),
            vec_d,
            vec_d,
            full((D, D)),
            vec_d,
            vec_d,
            vec_d,
            full((H, D)),
            full((H, D)),
            full((D, H)),
            vec_h,
            vec_h,
            vec_d,
        ],
        out_specs=pl.BlockSpec((1, LT, D), lambda i, j: (i, j, 0)),
        out_shape=jax.ShapeDtypeStruct((B, T, D), jnp.float32),
        scratch_shapes=[pltpu.VMEM((1, D), jnp.float32)],
        compiler_params=pltpu.CompilerParams(
            dimension_semantics=("parallel", "arbitrary"),
            vmem_limit_bytes=60000 * 1024,
        ),
        name="ssm_block_fused",
    )(x, a2, b2, ow16, ob2, n12, n22, w116, w216, w316, w1b2, w2b2, w3b2)
    return out


# speedup vs baseline: 1.4102x; 1.0080x over previous
import jax
import jax.numpy as jnp
from jax.experimental import pallas as pl
from jax.experimental.pallas import tpu as pltpu

EPS = 1e-5
LT = 256
HC = 2048

_TRANS_B = (((1,), (1,)), ((), ()))


def _dot_t(lhs, rhs):
    return jax.lax.dot_general(lhs, rhs, _TRANS_B,
                               preferred_element_type=jnp.float32)


def _make_body(D, H):
    def _body(x_ref, a_ref, b_ref, ow_ref, ob_ref, n1_ref, n2_ref,
              w1_ref, w2_ref, w3_ref, w1b_ref, w2b_ref, w3b_ref,
              o_ref, carry_ref):
        t = pl.program_id(1)

        @pl.when(t == 0)
        def _():
            carry_ref[...] = jnp.zeros_like(carry_ref)

        xb = x_ref[0]

        rms1 = jax.lax.rsqrt(jnp.mean(xb * xb, axis=-1, keepdims=True) + EPS)
        y = (xb * rms1) * (n1_ref[...] * b_ref[...])

        at = jnp.tanh(a_ref[...])
        row8 = jax.lax.broadcasted_iota(jnp.int32, (8, 1), 0)
        head = y[0:8] + jnp.where(row8 == 0, jnp.float32(1.0),
                                  jnp.float32(0.0)) * (at * carry_ref[...])
        y = jnp.concatenate([head, y[8:]], axis=0)
        p = at
        k = 1
        while k < LT:
            shifted = jnp.concatenate(
                [jnp.zeros((k, D), jnp.float32), y[: LT - k]], axis=0)
            y = y + p * shifted
            p = p * p
            k *= 2
        carry_ref[...] = y[LT - 1: LT]

        ssm = _dot_t(y.astype(jnp.bfloat16), ow_ref[...])
        x1 = xb + ssm + ob_ref[...]

        rms2 = jax.lax.rsqrt(jnp.mean(x1 * x1, axis=-1, keepdims=True) + EPS)
        x1n = ((x1 * rms2) * n2_ref[...]).astype(jnp.bfloat16)

        o = x1
        for hc in range(H // HC):
            rs = pl.ds(hc * HC, HC)
            cs = pl.ds(hc * HC, HC)
            u = _dot_t(x1n, w1_ref[rs, :]) + w1b_ref[:, cs]
            g = _dot_t(x1n, w2_ref[rs, :]) + w2b_ref[:, cs]
            h = (jax.nn.silu(g) * u).astype(jnp.bfloat16)
            o = o + _dot_t(h, w3_ref[:, cs])
        o_ref[0] = o + w3b_ref[...]

    return _body


def kernel(x, a, b, out_w, out_b, n1_w, n2_w,
           w1_w, w1_b, w2_w, w2_b, w3_w, w3_b):
    B, T, D = x.shape
    H = w1_w.shape[0]
    NT = T // LT

    ow16 = out_w.astype(jnp.bfloat16)
    w116 = w1_w.astype(jnp.bfloat16)
    w216 = w2_w.astype(jnp.bfloat16)
    w316 = w3_w.astype(jnp.bfloat16)

    a2 = a.reshape(1, D)
    b2 = b.reshape(1, D)
    ob2 = out_b.reshape(1, D)
    n12 = n1_w.reshape(1, D)
    n22 = n2_w.reshape(1, D)
    w1b2 = w1_b.reshape(1, H)
    w2b2 = w2_b.reshape(1, H)
    w3b2 = w3_b.reshape(1, D)

    vec_d = pl.BlockSpec((1, D), lambda i, j: (0, 0))
    vec_h = pl.BlockSpec((1, H), lambda i, j: (0, 0))
    full = lambda shape: pl.BlockSpec(shape, lambda i, j: (0, 0))

    grid = (B, NT)
    out = pl.pallas_call(
        _make_body(D, H),
        grid=grid,
        in_specs=[
            pl.BlockSpec((1, LT, D), lambda i, j: (i, j, 0)),
            vec_d,
            vec_d,
            full((D, D)),
            vec_d,
            vec_d,
            vec_d,
            full((H, D)),
            full((H, D)),
            full((D, H)),
            vec_h,
            vec_h,
            vec_d,
        ],
        out_specs=pl.BlockSpec((1, LT, D), lambda i, j: (i, j, 0)),
        out_shape=jax.ShapeDtypeStruct((B, T, D), jnp.float32),
        scratch_shapes=[pltpu.VMEM((1, D), jnp.float32)],
        compiler_params=pltpu.CompilerParams(
            dimension_semantics=("parallel", "arbitrary"),
            vmem_limit_bytes=60000 * 1024,
        ),
        name="ssm_block_fused",
    )(x, a2, b2, ow16, ob2, n12, n22, w116, w216, w316, w1b2, w2b2, w3b2)
    return out


# speedup vs baseline: 1.4165x; 1.0045x over previous
import jax
import jax.numpy as jnp
from jax.experimental import pallas as pl
from jax.experimental.pallas import tpu as pltpu

EPS = 1e-5
LT = 256
HC = 4096

_TRANS_B = (((1,), (1,)), ((), ()))


def _dot_t(lhs, rhs):
    return jax.lax.dot_general(lhs, rhs, _TRANS_B,
                               preferred_element_type=jnp.float32)


def _make_body(D, H):
    def _body(x_ref, a_ref, b_ref, ow_ref, ob_ref, n1_ref, n2_ref,
              w1_ref, w2_ref, w3_ref, w1b_ref, w2b_ref, w3b_ref,
              o_ref, carry_ref):
        t = pl.program_id(1)

        @pl.when(t == 0)
        def _():
            carry_ref[...] = jnp.zeros_like(carry_ref)

        xb = x_ref[0]

        rms1 = jax.lax.rsqrt(jnp.mean(xb * xb, axis=-1, keepdims=True) + EPS)
        y = (xb * rms1) * (n1_ref[...] * b_ref[...])

        at = jnp.tanh(a_ref[...])
        row8 = jax.lax.broadcasted_iota(jnp.int32, (8, 1), 0)
        head = y[0:8] + jnp.where(row8 == 0, jnp.float32(1.0),
                                  jnp.float32(0.0)) * (at * carry_ref[...])
        y = jnp.concatenate([head, y[8:]], axis=0)
        p = at
        k = 1
        while k < LT:
            shifted = jnp.concatenate(
                [jnp.zeros((k, D), jnp.float32), y[: LT - k]], axis=0)
            y = y + p * shifted
            p = p * p
            k *= 2
        carry_ref[...] = y[LT - 1: LT]

        ssm = _dot_t(y.astype(jnp.bfloat16), ow_ref[...])
        x1 = xb + ssm + ob_ref[...]

        rms2 = jax.lax.rsqrt(jnp.mean(x1 * x1, axis=-1, keepdims=True) + EPS)
        x1n = ((x1 * rms2) * n2_ref[...]).astype(jnp.bfloat16)

        o = x1
        for hc in range(H // HC):
            rs = pl.ds(hc * HC, HC)
            cs = pl.ds(hc * HC, HC)
            u = _dot_t(x1n, w1_ref[rs, :]) + w1b_ref[:, cs]
            g = _dot_t(x1n, w2_ref[rs, :]) + w2b_ref[:, cs]
            h = (jax.nn.silu(g) * u).astype(jnp.bfloat16)
            o = o + _dot_t(h, w3_ref[:, cs])
        o_ref[0] = o + w3b_ref[...]

    return _body


def kernel(x, a, b, out_w, out_b, n1_w, n2_w,
           w1_w, w1_b, w2_w, w2_b, w3_w, w3_b):
    B, T, D = x.shape
    H = w1_w.shape[0]
    NT = T // LT

    ow16 = out_w.astype(jnp.bfloat16)
    w116 = w1_w.astype(jnp.bfloat16)
    w216 = w2_w.astype(jnp.bfloat16)
    w316 = w3_w.astype(jnp.bfloat16)

    a2 = a.reshape(1, D)
    b2 = b.reshape(1, D)
    ob2 = out_b.reshape(1, D)
    n12 = n1_w.reshape(1, D)
    n22 = n2_w.reshape(1, D)
    w1b2 = w1_b.reshape(1, H)
    w2b2 = w2_b.reshape(1, H)
    w3b2 = w3_b.reshape(1, D)

    vec_d = pl.BlockSpec((1, D), lambda i, j: (0, 0))
    vec_h = pl.BlockSpec((1, H), lambda i, j: (0, 0))
    full = lambda shape: pl.BlockSpec(shape, lambda i, j: (0, 0))

    grid = (B, NT)
    out = pl.pallas_call(
        _make_body(D, H),
        grid=grid,
        in_specs=[
            pl.BlockSpec((1, LT, D), lambda i, j: (i, j, 0)),
            vec_d,
            vec_d,
            full((D, D)),
            vec_d,
            vec_d,
            vec_d,
            full((H, D)),
            full((H, D)),
            full((D, H)),
            vec_h,
            vec_h,
            vec_d,
        ],
        out_specs=pl.BlockSpec((1, LT, D), lambda i, j: (i, j, 0)),
        out_shape=jax.ShapeDtypeStruct((B, T, D), jnp.float32),
        scratch_shapes=[pltpu.VMEM((1, D), jnp.float32)],
        compiler_params=pltpu.CompilerParams(
            dimension_semantics=("parallel", "arbitrary"),
            vmem_limit_bytes=60000 * 1024,
        ),
        name="ssm_block_fused",
    )(x, a2, b2, ow16, ob2, n12, n22, w116, w216, w316, w1b2, w2b2, w3b2)
    return out


# speedup vs baseline: 1.4470x; 1.0215x over previous
import jax
import jax.numpy as jnp
from jax.experimental import pallas as pl
from jax.experimental.pallas import tpu as pltpu

EPS = 1e-5
LT = 512
HC = 4096

_TRANS_B = (((1,), (1,)), ((), ()))


def _dot_t(lhs, rhs):
    return jax.lax.dot_general(lhs, rhs, _TRANS_B,
                               preferred_element_type=jnp.float32)


def _make_body(D, H):
    def _body(x_ref, a_ref, b_ref, ow_ref, ob_ref, n1_ref, n2_ref,
              w1_ref, w2_ref, w3_ref, w1b_ref, w2b_ref, w3b_ref,
              o_ref, carry_ref):
        t = pl.program_id(1)

        @pl.when(t == 0)
        def _():
            carry_ref[...] = jnp.zeros_like(carry_ref)

        xb = x_ref[0]

        rms1 = jax.lax.rsqrt(jnp.mean(xb * xb, axis=-1, keepdims=True) + EPS)
        y = (xb * rms1) * (n1_ref[...] * b_ref[...])

        at = jnp.tanh(a_ref[...])
        row8 = jax.lax.broadcasted_iota(jnp.int32, (8, 1), 0)
        head = y[0:8] + jnp.where(row8 == 0, jnp.float32(1.0),
                                  jnp.float32(0.0)) * (at * carry_ref[...])
        y = jnp.concatenate([head, y[8:]], axis=0)
        p = at
        k = 1
        while k < LT:
            shifted = jnp.concatenate(
                [jnp.zeros((k, D), jnp.float32), y[: LT - k]], axis=0)
            y = y + p * shifted
            p = p * p
            k *= 2
        carry_ref[...] = y[LT - 1: LT]

        ssm = _dot_t(y.astype(jnp.bfloat16), ow_ref[...])
        x1 = xb + ssm + ob_ref[...]

        rms2 = jax.lax.rsqrt(jnp.mean(x1 * x1, axis=-1, keepdims=True) + EPS)
        x1n = ((x1 * rms2) * n2_ref[...]).astype(jnp.bfloat16)

        o = x1
        for hc in range(H // HC):
            rs = pl.ds(hc * HC, HC)
            cs = pl.ds(hc * HC, HC)
            u = _dot_t(x1n, w1_ref[rs, :]) + w1b_ref[:, cs]
            g = _dot_t(x1n, w2_ref[rs, :]) + w2b_ref[:, cs]
            h = (jax.nn.silu(g) * u).astype(jnp.bfloat16)
            o = o + _dot_t(h, w3_ref[:, cs])
        o_ref[0] = o + w3b_ref[...]

    return _body


def kernel(x, a, b, out_w, out_b, n1_w, n2_w,
           w1_w, w1_b, w2_w, w2_b, w3_w, w3_b):
    B, T, D = x.shape
    H = w1_w.shape[0]
    NT = T // LT

    ow16 = out_w.astype(jnp.bfloat16)
    w116 = w1_w.astype(jnp.bfloat16)
    w216 = w2_w.astype(jnp.bfloat16)
    w316 = w3_w.astype(jnp.bfloat16)

    a2 = a.reshape(1, D)
    b2 = b.reshape(1, D)
    ob2 = out_b.reshape(1, D)
    n12 = n1_w.reshape(1, D)
    n22 = n2_w.reshape(1, D)
    w1b2 = w1_b.reshape(1, H)
    w2b2 = w2_b.reshape(1, H)
    w3b2 = w3_b.reshape(1, D)

    vec_d = pl.BlockSpec((1, D), lambda i, j: (0, 0))
    vec_h = pl.BlockSpec((1, H), lambda i, j: (0, 0))
    full = lambda shape: pl.BlockSpec(shape, lambda i, j: (0, 0))

    grid = (B, NT)
    out = pl.pallas_call(
        _make_body(D, H),
        grid=grid,
        in_specs=[
            pl.BlockSpec((1, LT, D), lambda i, j: (i, j, 0)),
            vec_d,
            vec_d,
            full((D, D)),
            vec_d,
            vec_d,
            vec_d,
            full((H, D)),
            full((H, D)),
            full((D, H)),
            vec_h,
            vec_h,
            vec_d,
        ],
        out_specs=pl.BlockSpec((1, LT, D), lambda i, j: (i, j, 0)),
        out_shape=jax.ShapeDtypeStruct((B, T, D), jnp.float32),
        scratch_shapes=[pltpu.VMEM((1, D), jnp.float32)],
        compiler_params=pltpu.CompilerParams(
            dimension_semantics=("parallel", "arbitrary"),
            vmem_limit_bytes=60000 * 1024,
        ),
        name="ssm_block_fused",
    )(x, a2, b2, ow16, ob2, n12, n22, w116, w216, w316, w1b2, w2b2, w3b2)
    return out
